# baseline (speedup 1.0000x reference)
import functools

import jax
import jax.numpy as jnp
from jax import lax
from jax.experimental import pallas as pl
from jax.experimental.pallas import tpu as pltpu
from jax.experimental.pallas import tpu_sc as plsc

_EMBED = 32
_IDX_CHUNK = 128


def _dot_kernel(uid_hbm, mid_hbm, utab_hbm, mtab_hbm, out_hbm,
                uid_v, mid_v, urows_v, mrows_v, out_v, sem,
                *, b_per_w, num_cores):
    wid = lax.axis_index("s") * num_cores + lax.axis_index("c")
    base = wid * b_per_w

    pltpu.sync_copy(uid_hbm.at[pl.ds(base, b_per_w)], uid_v)
    pltpu.sync_copy(mid_hbm.at[pl.ds(base, b_per_w)], mid_v)

    n_chunks = b_per_w // _IDX_CHUNK
    copies = []
    for j in range(n_chunks):
        sl = pl.ds(j * _IDX_CHUNK, _IDX_CHUNK)
        copies.append(pltpu.async_copy(
            utab_hbm.at[uid_v.at[sl]], urows_v.at[sl], sem))
        copies.append(pltpu.async_copy(
            mtab_hbm.at[mid_v.at[sl]], mrows_v.at[sl], sem))
    for c in copies:
        c.wait()

    lane = lax.iota(jnp.int32, 16)

    def body(g, _):
        row0 = g * 16
        acc = jnp.zeros((16,), jnp.float32)
        for k in range(16):
            row = row0 + k
            v = (urows_v[row, pl.ds(0, 16)] * mrows_v[row, pl.ds(0, 16)]
                 + urows_v[row, pl.ds(16, 16)] * mrows_v[row, pl.ds(16, 16)])
            acc = jnp.where(lane == k, jnp.sum(v), acc)
        out_v[pl.ds(row0, 16)] = acc
        return 0

    lax.fori_loop(0, b_per_w // 16, body, 0)

    pltpu.sync_copy(out_v, out_hbm.at[pl.ds(base, b_per_w)])


def kernel(user_ids, movie_ids, user_table, movie_table):
    batch = user_ids.shape[0]
    info = plsc.get_sparse_core_info()
    nw = info.num_cores * info.num_subcores
    b_per_w = batch // nw
    mesh = plsc.VectorSubcoreMesh(core_axis_name="c", subcore_axis_name="s")

    run = pl.kernel(
        functools.partial(_dot_kernel, b_per_w=b_per_w,
                          num_cores=info.num_cores),
        mesh=mesh,
        compiler_params=pltpu.CompilerParams(
            needs_layout_passes=False, use_tc_tiling_on_sc=False),
        out_type=jax.ShapeDtypeStruct((batch,), jnp.float32),
        scratch_types=[
            pltpu.VMEM((b_per_w,), jnp.int32),
            pltpu.VMEM((b_per_w,), jnp.int32),
            pltpu.VMEM((b_per_w, _EMBED), jnp.float32),
            pltpu.VMEM((b_per_w, _EMBED), jnp.float32),
            pltpu.VMEM((b_per_w,), jnp.float32),
            pltpu.SemaphoreType.DMA,
        ],
    )
    return run(user_ids.astype(jnp.int32), movie_ids.astype(jnp.int32),
               user_table, movie_table)
